# baseline (speedup 1.0000x reference)
import functools

import jax
import jax.numpy as jnp
from jax import lax
from jax.experimental import pallas as pl
from jax.experimental.pallas import tpu as pltpu
from jax.experimental.pallas import tpu_sc as plsc

LANES = 16


@functools.lru_cache(maxsize=None)
def _make_sc_embed(N, D, S, V, CH, NB):
    info = plsc.get_sparse_core_info()
    NC, NS = info.num_cores, info.num_subcores
    NW = NC * NS
    n_w = N // NW
    J = n_w // CH
    R = V * S
    H = D // LANES
    assert N % (NW * CH) == 0 and CH % LANES == 0
    assert D % LANES == 0 and CH % S == 0 and J % NB == 0 and J // NB >= 2

    mesh = plsc.VectorSubcoreMesh(core_axis_name="c", subcore_axis_name="s")

    @functools.partial(
        pl.kernel,
        mesh=mesh,
        out_type=jax.ShapeDtypeStruct((N * D,), jnp.float32),
        scratch_types=(
            [pltpu.VMEM((V, D), jnp.float32),
             pltpu.VMEM((S, D), jnp.float32),
             pltpu.VMEM((R * D,), jnp.float32),
             pltpu.VMEM((n_w,), jnp.int32)]
            + [pltpu.VMEM((CH * D,), jnp.float32) for _ in range(NB)]
            + [pltpu.SemaphoreType.DMA for _ in range(NB)]
        ),
        compiler_params=pltpu.CompilerParams(use_tc_tiling_on_sc=False,
                                             needs_layout_passes=False),
    )
    def k(tok_hbm, pos_hbm, xf_hbm, out_hbm, tok_v, pos_v, fus_v, idx_v,
          *rest):
        bufs = rest[:NB]
        sems = rest[NB:2 * NB]
        wid = lax.axis_index("s") * NC + lax.axis_index("c")
        base = wid * n_w
        pltpu.sync_copy(tok_hbm, tok_v)
        pltpu.sync_copy(pos_hbm, pos_v)
        pltpu.sync_copy(xf_hbm.at[pl.ds(base, n_w)], idx_v)
        ps = [[pos_v[s, pl.ds(h * LANES, LANES)] for h in range(H)]
              for s in range(S)]
        for v in range(V):
            th = [tok_v[v, pl.ds(h * LANES, LANES)] for h in range(H)]
            for s in range(S):
                for h in range(H):
                    fus_v[pl.ds((v * S + s) * D + h * LANES, LANES)] = (
                        th[h] + ps[s][h])
        lane = lax.broadcasted_iota(jnp.int32, (LANES,), 0)
        pat = (lane % S) * D
        sbase = lane * D

        def fill(j, b):
            for g in range(CH // LANES):
                raw = idx_v[pl.ds(j * CH + g * LANES, LANES)]
                ga = raw * (S * D) + pat
                sb = sbase + g * LANES * D
                for c in range(D):
                    plsc.store_scatter(bufs[b], [sb + c],
                                       plsc.load_gather(fus_v, [ga + c]))

        def store(j, b):
            return pltpu.async_copy(
                bufs[b], out_hbm.at[pl.ds((base + j * CH) * D, CH * D)],
                sems[b])

        for b in range(NB):
            fill(b, b)
            store(b, b)

        @pl.loop(NB, J, step=NB)
        def _(j0):
            for b in range(NB):
                pltpu.make_async_copy(
                    bufs[b], out_hbm.at[pl.ds(base * D, CH * D)],
                    sems[b]).wait()
                fill(j0 + b, b)
                store(j0 + b, b)

        for b in range(NB):
            pltpu.make_async_copy(
                bufs[b], out_hbm.at[pl.ds(base * D, CH * D)], sems[b]).wait()

    return k


def kernel(x, token_table, pos_table):
    B, S = x.shape
    V, D = token_table.shape
    N = B * S
    xf = x.reshape(N)
    out = _make_sc_embed(N, D, S, V, 128, 2)(token_table, pos_table[:S], xf)
    return out.reshape(B, S, D)


# speedup vs baseline: 1.3280x; 1.3280x over previous
import functools

import jax
import jax.numpy as jnp
from jax import lax
from jax.experimental import pallas as pl
from jax.experimental.pallas import tpu as pltpu
from jax.experimental.pallas import tpu_sc as plsc

LANES = 16


@functools.lru_cache(maxsize=None)
def _make_sc_embed(N, D, S, V, CH, NB):
    info = plsc.get_sparse_core_info()
    NC, NS = info.num_cores, info.num_subcores
    NW = NC * NS
    n_w = N // NW
    J = n_w // CH
    R = V * S
    H = D // LANES
    W = D + 1
    assert N % (NW * CH) == 0 and CH % LANES == 0
    assert D % LANES == 0 and CH % S == 0 and J % NB == 0 and J // NB >= 2

    mesh = plsc.VectorSubcoreMesh(core_axis_name="c", subcore_axis_name="s")

    @functools.partial(
        pl.kernel,
        mesh=mesh,
        out_type=jax.ShapeDtypeStruct((N, D), jnp.float32),
        scratch_types=(
            [pltpu.VMEM((V, D), jnp.float32),
             pltpu.VMEM((S, D), jnp.float32),
             pltpu.VMEM((R * W,), jnp.float32),
             pltpu.VMEM((n_w,), jnp.int32)]
            + [pltpu.VMEM((CH, W), jnp.float32) for _ in range(NB)]
            + [pltpu.SemaphoreType.DMA for _ in range(NB)]
        ),
        compiler_params=pltpu.CompilerParams(use_tc_tiling_on_sc=False,
                                             needs_layout_passes=False),
    )
    def k(tok_hbm, pos_hbm, xf_hbm, out_hbm, tok_v, pos_v, fus_v, idx_v,
          *rest):
        bufs = rest[:NB]
        sems = rest[NB:2 * NB]
        wid = lax.axis_index("s") * NC + lax.axis_index("c")
        base = wid * n_w
        pltpu.sync_copy(tok_hbm, tok_v)
        pltpu.sync_copy(pos_hbm, pos_v)
        pltpu.sync_copy(xf_hbm.at[pl.ds(base, n_w)], idx_v)
        ps = [[pos_v[s, pl.ds(h * LANES, LANES)] for h in range(H)]
              for s in range(S)]
        for v in range(V):
            th = [tok_v[v, pl.ds(h * LANES, LANES)] for h in range(H)]
            for s in range(S):
                for h in range(H):
                    fus_v[pl.ds((v * S + s) * W + h * LANES, LANES)] = (
                        th[h] + ps[s][h])
        lane = lax.broadcasted_iota(jnp.int32, (LANES,), 0)
        pat = (lane % S) * W

        def fill(j, b):
            for g in range(CH // LANES):
                raw = idx_v[pl.ds(j * CH + g * LANES, LANES)]
                ga = raw * (S * W) + pat
                rows = lane + g * LANES
                for c in range(D):
                    plsc.store_scatter(
                        bufs[b], [rows, jnp.full((LANES,), c, jnp.int32)],
                        plsc.load_gather(fus_v, [ga + c]))

        def store(j, b):
            return pltpu.async_copy(
                bufs[b].at[:, pl.ds(0, D)],
                out_hbm.at[pl.ds(base + j * CH, CH)], sems[b])

        for b in range(NB):
            fill(b, b)
            store(b, b)

        @pl.loop(NB, J, step=NB)
        def _(j0):
            for b in range(NB):
                pltpu.make_async_copy(
                    bufs[b].at[:, pl.ds(0, D)],
                    out_hbm.at[pl.ds(base, CH)], sems[b]).wait()
                fill(j0 + b, b)
                store(j0 + b, b)

        for b in range(NB):
            pltpu.make_async_copy(
                bufs[b].at[:, pl.ds(0, D)],
                out_hbm.at[pl.ds(base, CH)], sems[b]).wait()

    return k


def kernel(x, token_table, pos_table):
    B, S = x.shape
    V, D = token_table.shape
    N = B * S
    xf = x.reshape(N)
    out = _make_sc_embed(N, D, S, V, 128, 2)(token_table, pos_table[:S], xf)
    return out.reshape(B, S, D)


# speedup vs baseline: 2.3227x; 1.7490x over previous
import functools

import jax
import jax.numpy as jnp
from jax import lax
from jax.experimental import pallas as pl
from jax.experimental.pallas import tpu as pltpu
from jax.experimental.pallas import tpu_sc as plsc

LANES = 16


@functools.lru_cache(maxsize=None)
def _make_sc_embed(N, D, S, V, CH, NB):
    info = plsc.get_sparse_core_info()
    NC, NS = info.num_cores, info.num_subcores
    NW = NC * NS
    n_w = N // NW
    J = n_w // CH
    R = V * S
    H = D // LANES
    assert N % (NW * CH) == 0 and CH % LANES == 0 and CH <= 128
    assert D % LANES == 0 and CH % S == 0 and J >= 2 * NB

    mesh = plsc.VectorSubcoreMesh(core_axis_name="c", subcore_axis_name="s")

    @functools.partial(
        pl.kernel,
        mesh=mesh,
        out_type=jax.ShapeDtypeStruct((N, D), jnp.float32),
        scratch_types=(
            [pltpu.VMEM_SHARED((R, D), jnp.float32),
             pltpu.VMEM((V, D), jnp.float32),
             pltpu.VMEM((S, D), jnp.float32),
             pltpu.VMEM((R, D), jnp.float32),
             pltpu.VMEM((J, CH), jnp.int32)]
            + [pltpu.VMEM((CH, D), jnp.float32) for _ in range(NB)]
            + [pltpu.SemaphoreType.DMA for _ in range(2 * NB)]
        ),
        compiler_params=pltpu.CompilerParams(use_tc_tiling_on_sc=False,
                                             needs_layout_passes=False),
    )
    def k(tok_hbm, pos_hbm, xf_hbm, out_hbm, spm, tok_v, pos_v, fus_v,
          idx_v, *rest):
        bufs = rest[:NB]
        gsems = rest[NB:2 * NB]
        ssems = rest[2 * NB:3 * NB]
        sid = lax.axis_index("s")
        wid = sid * NC + lax.axis_index("c")
        pltpu.sync_copy(tok_hbm, tok_v)
        pltpu.sync_copy(pos_hbm, pos_v)
        pltpu.sync_copy(xf_hbm.at[pl.ds(wid * J, J)], idx_v)
        ps = [[pos_v[s, pl.ds(h * LANES, LANES)] for h in range(H)]
              for s in range(S)]
        for v in range(V):
            th = [tok_v[v, pl.ds(h * LANES, LANES)] for h in range(H)]
            for s in range(S):
                for h in range(H):
                    fus_v[v * S + s, pl.ds(h * LANES, LANES)] = th[h] + ps[s][h]
        @pl.when(sid == 0)
        def _():
            pltpu.sync_copy(fus_v, spm)
        plsc.subcore_barrier()
        lane = lax.broadcasted_iota(jnp.int32, (LANES,), 0)
        pat = lane % S
        for j in range(J):
            for g in range(CH // LANES):
                sl = pl.ds(g * LANES, LANES)
                idx_v[j, sl] = idx_v[j, sl] * S + pat
        base = wid * n_w

        def gather(j, b):
            return pltpu.async_copy(spm.at[idx_v.at[j]], bufs[b], gsems[b])

        def store(j, b):
            return pltpu.async_copy(
                bufs[b], out_hbm.at[pl.ds(base + j * CH, CH)], ssems[b])

        gd = [gather(b, b) for b in range(NB)]
        sd = [None] * NB
        for j in range(J):
            b = j % NB
            gd[b].wait()
            sd[b] = store(j, b)
            nxt = j + NB
            if nxt < J:
                sd[b].wait()
                gd[b] = gather(nxt, b)
        for b in range(NB):
            sd[b].wait()

    return k


def kernel(x, token_table, pos_table):
    B, S = x.shape
    V, D = token_table.shape
    CH = 128
    N = B * S
    xf = x.reshape(N // CH, CH)
    out = _make_sc_embed(N, D, S, V, CH, 2)(token_table, pos_table[:S], xf)
    return out.reshape(B, S, D)


# speedup vs baseline: 2.4337x; 1.0478x over previous
import functools

import jax
import jax.numpy as jnp
from jax import lax
from jax.experimental import pallas as pl
from jax.experimental.pallas import tpu as pltpu
from jax.experimental.pallas import tpu_sc as plsc

LANES = 16


@functools.lru_cache(maxsize=None)
def _make_sc_embed(N, D, S, V, CH, NB):
    info = plsc.get_sparse_core_info()
    NC, NS = info.num_cores, info.num_subcores
    NW = NC * NS
    B = N // S
    NQ = NW // S
    n_b = B // NQ
    J = n_b // CH
    H = D // LANES
    DH = D // 8
    W = D + 1
    R = V * S
    assert CH == 128 and D % 8 == 0 and B % (NQ * CH) == 0 and J % NB == 0

    mesh = plsc.VectorSubcoreMesh(core_axis_name="c", subcore_axis_name="s")

    @functools.partial(
        pl.kernel,
        mesh=mesh,
        out_type=jax.ShapeDtypeStruct((S, DH, B // CH, 8, CH), jnp.float32),
        scratch_types=(
            [pltpu.VMEM((V, D), jnp.float32),
             pltpu.VMEM((S, D), jnp.float32),
             pltpu.VMEM((R * W,), jnp.float32),
             pltpu.VMEM((n_b * S,), jnp.int32),
             pltpu.VMEM((n_b,), jnp.int32)]
            + [pltpu.VMEM((DH, 8, CH), jnp.float32) for _ in range(NB)]
            + [pltpu.SemaphoreType.DMA for _ in range(NB * DH)]
        ),
        compiler_params=pltpu.CompilerParams(use_tc_tiling_on_sc=False,
                                             needs_layout_passes=False),
    )
    def k(tok_hbm, pos_hbm, xf_hbm, out_hbm, tok_v, pos_v, fus_v, xblk_v,
          idx_v, *rest):
        bufs = rest[:NB]
        sems = rest[NB:NB + NB * DH]
        wid = lax.axis_index("s") * NC + lax.axis_index("c")
        sw = wid % S
        q = wid // S
        b0 = q * n_b
        pltpu.sync_copy(tok_hbm, tok_v)
        pltpu.sync_copy(pos_hbm, pos_v)
        pltpu.sync_copy(xf_hbm.at[pl.ds(b0 * S, n_b * S)], xblk_v)
        ps = [[pos_v[s, pl.ds(h * LANES, LANES)] for h in range(H)]
              for s in range(S)]
        for v in range(V):
            th = [tok_v[v, pl.ds(h * LANES, LANES)] for h in range(H)]
            for s in range(S):
                for h in range(H):
                    fus_v[pl.ds((v * S + s) * W + h * LANES, LANES)] = (
                        th[h] + ps[s][h])
        lane = lax.broadcasted_iota(jnp.int32, (LANES,), 0)

        @pl.loop(0, n_b // (16 * LANES))
        def _(i0):
            for g16 in range(16):
                g = i0 * 16 + g16
                raw = plsc.load_gather(
                    xblk_v, [(g * LANES + lane) * S + sw])
                idx_v[pl.ds(g * LANES, LANES)] = raw * (S * W) + sw * W

        def fill(j, b):
            for g in range(CH // LANES):
                ga = idx_v[pl.ds(j * CH + g * LANES, LANES)]
                for d in range(D):
                    bufs[b][d // 8, d % 8, pl.ds(g * LANES, LANES)] = (
                        plsc.load_gather(fus_v, [ga + d]))

        def store(j, b):
            for dh in range(DH):
                pltpu.async_copy(bufs[b].at[dh],
                                 out_hbm.at[sw, dh, q * J + j],
                                 sems[b * DH + dh])

        def wait(b):
            for dh in range(DH):
                pltpu.make_async_copy(bufs[b].at[dh],
                                      out_hbm.at[0, 0, 0],
                                      sems[b * DH + dh]).wait()

        for b in range(NB):
            fill(b, b)
            store(b, b)

        @pl.loop(NB, J, step=NB)
        def _(j0):
            for b in range(NB):
                wait(b)
                fill(j0 + b, b)
                store(j0 + b, b)

        for b in range(NB):
            wait(b)

    return k


def kernel(x, token_table, pos_table):
    B, S = x.shape
    V, D = token_table.shape
    N = B * S
    xf = x.reshape(N)
    p = _make_sc_embed(N, D, S, V, 128, 2)(token_table, pos_table[:S], xf)
    return p.transpose(2, 4, 0, 1, 3).reshape(B, S, D)


# speedup vs baseline: 2.7338x; 1.1233x over previous
import functools

import jax
import jax.numpy as jnp
from jax import lax
from jax.experimental import pallas as pl
from jax.experimental.pallas import tpu as pltpu
from jax.experimental.pallas import tpu_sc as plsc

LANES = 16


@functools.lru_cache(maxsize=None)
def _make_sc_embed(N, D, S, V, CH, NB):
    info = plsc.get_sparse_core_info()
    NC, NS = info.num_cores, info.num_subcores
    NW = NC * NS
    B = N // S
    NQ = NW // S
    n_b = B // NQ
    J = n_b // CH
    H = D // LANES
    DH = D // 8
    W = D + 1
    R = V * S
    assert CH == 128 and D % 8 == 0 and B % (NQ * CH) == 0 and J % NB == 0

    mesh = plsc.VectorSubcoreMesh(core_axis_name="c", subcore_axis_name="s")

    @functools.partial(
        pl.kernel,
        mesh=mesh,
        out_type=jax.ShapeDtypeStruct((S, DH, B // CH, 8, CH), jnp.float32),
        scratch_types=(
            [pltpu.VMEM((V, D), jnp.float32),
             pltpu.VMEM((S, D), jnp.float32),
             pltpu.VMEM((R * W,), jnp.float32),
             pltpu.VMEM((n_b * S,), jnp.int32),
             pltpu.VMEM((n_b,), jnp.int32)]
            + [pltpu.VMEM((DH, 8, CH), jnp.float32) for _ in range(NB)]
            + [pltpu.SemaphoreType.DMA for _ in range(NB * DH)]
        ),
        compiler_params=pltpu.CompilerParams(use_tc_tiling_on_sc=False,
                                             needs_layout_passes=False),
    )
    def k(tok_hbm, pos_hbm, xf_hbm, out_hbm, tok_v, pos_v, fus_v, xblk_v,
          idx_v, *rest):
        bufs = rest[:NB]
        sems = rest[NB:NB + NB * DH]
        wid = lax.axis_index("s") * NC + lax.axis_index("c")
        sw = wid % S
        q = wid // S
        b0 = q * n_b
        pltpu.sync_copy(tok_hbm, tok_v)
        pltpu.sync_copy(pos_hbm, pos_v)
        pltpu.sync_copy(xf_hbm.at[pl.ds(b0 * S, n_b * S)], xblk_v)
        ps = [[pos_v[s, pl.ds(h * LANES, LANES)] for h in range(H)]
              for s in range(S)]
        for v in range(V):
            th = [tok_v[v, pl.ds(h * LANES, LANES)] for h in range(H)]
            for s in range(S):
                for h in range(H):
                    fus_v[pl.ds((s * V + v) * W + h * LANES, LANES)] = (
                        th[h] + ps[s][h])
        lane = lax.broadcasted_iota(jnp.int32, (LANES,), 0)

        @pl.loop(0, n_b // (16 * LANES))
        def _(i0):
            for g16 in range(16):
                g = i0 * 16 + g16
                raw = plsc.load_gather(
                    xblk_v, [(g * LANES + lane) * S + sw])
                idx_v[pl.ds(g * LANES, LANES)] = (raw + sw * V) * W

        def fill(j, b):
            for g in range(CH // LANES):
                ga = idx_v[pl.ds(j * CH + g * LANES, LANES)]
                for d in range(D):
                    bufs[b][d // 8, d % 8, pl.ds(g * LANES, LANES)] = (
                        plsc.load_gather(fus_v, [ga + d]))

        def store(j, b):
            for dh in range(DH):
                pltpu.async_copy(bufs[b].at[dh],
                                 out_hbm.at[sw, dh, q * J + j],
                                 sems[b * DH + dh])

        def wait(b):
            for dh in range(DH):
                pltpu.make_async_copy(bufs[b].at[dh],
                                      out_hbm.at[0, 0, 0],
                                      sems[b * DH + dh]).wait()

        for b in range(NB):
            fill(b, b)
            store(b, b)

        @pl.loop(NB, J, step=NB)
        def _(j0):
            for b in range(NB):
                wait(b)
                fill(j0 + b, b)
                store(j0 + b, b)

        for b in range(NB):
            wait(b)

    return k


def kernel(x, token_table, pos_table):
    B, S = x.shape
    V, D = token_table.shape
    N = B * S
    xf = x.reshape(N)
    p = _make_sc_embed(N, D, S, V, 128, 2)(token_table, pos_table[:S], xf)
    return p.transpose(2, 4, 0, 1, 3).reshape(B, S, D)
